# baseline (speedup 1.0000x reference)
import functools

import jax
import jax.numpy as jnp
from jax import lax
from jax.experimental import pallas as pl
from jax.experimental.pallas import tpu as pltpu
from jax.experimental.pallas import tpu_sc as plsc

_N = 10000
_E = 160000
_H = 128
_NP = 10240
_EP = 163840
_NW = 32
_CHUNK = 128
_RPW = _EP // _NW
_NCH = _RPW // _CHUNK
_ZR = _NP // 16
_BLK = 512

def _mesh():
    return plsc.VectorSubcoreMesh(core_axis_name="c", subcore_axis_name="s")


def _dot(a, b):
    return lax.dot_general(a, b, (((1,), (0,)), ((), ())),
                           preferred_element_type=jnp.float32)


def _mlp2_body(x_ref, w1_ref, b1_ref, w2_ref, b2_ref, o_ref):
    t = jnp.maximum(_dot(x_ref[...], w1_ref[...]) + b1_ref[...], 0.0)
    o_ref[...] = _dot(t, w2_ref[...]) + b2_ref[...]


def _mlp2_rows(xx, w1, b1, w2, b2):
    r, k = xx.shape
    hh = w1.shape[1]
    oo = w2.shape[1]
    return pl.pallas_call(
        _mlp2_body,
        grid=(r // _BLK,),
        in_specs=[pl.BlockSpec((_BLK, k), lambda i: (i, 0)),
                  pl.BlockSpec((k, hh), lambda i: (0, 0)),
                  pl.BlockSpec((1, hh), lambda i: (0, 0)),
                  pl.BlockSpec((hh, oo), lambda i: (0, 0)),
                  pl.BlockSpec((1, oo), lambda i: (0, 0))],
        out_specs=pl.BlockSpec((_BLK, oo), lambda i: (i, 0)),
        out_shape=jax.ShapeDtypeStruct((r, oo), jnp.float32),
        compiler_params=pltpu.CompilerParams(
            dimension_semantics=("parallel",)),
    )(xx, w1, b1.reshape(1, -1), w2, b2.reshape(1, -1))


def _dual_mm_body(h_ref, wa_ref, wb_ref, o1_ref, o2_ref):
    h = h_ref[...]
    o1_ref[...] = _dot(h, wa_ref[...])
    o2_ref[...] = _dot(h, wb_ref[...])


def _dual_mm(h, wa, wb):
    r = h.shape[0]
    return pl.pallas_call(
        _dual_mm_body,
        grid=(r // _BLK,),
        in_specs=[pl.BlockSpec((_BLK, _H), lambda i: (i, 0)),
                  pl.BlockSpec((_H, _H), lambda i: (0, 0)),
                  pl.BlockSpec((_H, _H), lambda i: (0, 0))],
        out_specs=[pl.BlockSpec((_BLK, _H), lambda i: (i, 0)),
                   pl.BlockSpec((_BLK, _H), lambda i: (i, 0))],
        out_shape=[jax.ShapeDtypeStruct((r, _H), jnp.float32),
                   jax.ShapeDtypeStruct((r, _H), jnp.float32)],
        compiler_params=pltpu.CompilerParams(
            dimension_semantics=("parallel",)),
    )(h, wa, wb)


def _edge_post_body(g1_ref, g2_ref, e_ref, wc_ref, b1_ref, w2_ref, b2_ref,
                    m_ref, eo_ref):
    e = e_ref[...]
    pre = (g1_ref[...] + g2_ref[...]
           + _dot(e, wc_ref[...]) + b1_ref[...])
    m = _dot(jnp.maximum(pre, 0.0), w2_ref[...]) + b2_ref[...]
    m_ref[...] = m
    eo_ref[...] = e + m


def _edge_post(g1, g2, e, wc, b1, w2, b2):
    r = e.shape[0]
    return pl.pallas_call(
        _edge_post_body,
        grid=(r // _BLK,),
        in_specs=[pl.BlockSpec((_BLK, _H), lambda i: (i, 0)),
                  pl.BlockSpec((_BLK, _H), lambda i: (i, 0)),
                  pl.BlockSpec((_BLK, _H), lambda i: (i, 0)),
                  pl.BlockSpec((_H, _H), lambda i: (0, 0)),
                  pl.BlockSpec((1, _H), lambda i: (0, 0)),
                  pl.BlockSpec((_H, _H), lambda i: (0, 0)),
                  pl.BlockSpec((1, _H), lambda i: (0, 0))],
        out_specs=[pl.BlockSpec((_BLK, _H), lambda i: (i, 0)),
                   pl.BlockSpec((_BLK, _H), lambda i: (i, 0))],
        out_shape=[jax.ShapeDtypeStruct((r, _H), jnp.float32),
                   jax.ShapeDtypeStruct((r, _H), jnp.float32)],
        compiler_params=pltpu.CompilerParams(
            dimension_semantics=("parallel",)),
    )(g1, g2, e, wc, b1.reshape(1, -1), w2, b2.reshape(1, -1))


def _node_post_body(h_ref, p0_ref, p1_ref, wa_ref, wb_ref, b1_ref,
                    w2_ref, b2_ref, o_ref):
    h = h_ref[...]
    agg = p0_ref[...] + p1_ref[...]
    t = jnp.maximum(_dot(h, wa_ref[...]) + _dot(agg, wb_ref[...]) + b1_ref[...],
                    0.0)
    o_ref[...] = h + _dot(t, w2_ref[...]) + b2_ref[...]


def _node_post(h, p0, p1, wa, wb, b1, w2, b2):
    r = h.shape[0]
    return pl.pallas_call(
        _node_post_body,
        grid=(r // _BLK,),
        in_specs=[pl.BlockSpec((_BLK, _H), lambda i: (i, 0)),
                  pl.BlockSpec((_BLK, _H), lambda i: (i, 0)),
                  pl.BlockSpec((_BLK, _H), lambda i: (i, 0)),
                  pl.BlockSpec((_H, _H), lambda i: (0, 0)),
                  pl.BlockSpec((_H, _H), lambda i: (0, 0)),
                  pl.BlockSpec((1, _H), lambda i: (0, 0)),
                  pl.BlockSpec((_H, _H), lambda i: (0, 0)),
                  pl.BlockSpec((1, _H), lambda i: (0, 0))],
        out_specs=pl.BlockSpec((_BLK, _H), lambda i: (i, 0)),
        out_shape=jax.ShapeDtypeStruct((r, _H), jnp.float32),
        compiler_params=pltpu.CompilerParams(
            dimension_semantics=("parallel",)),
    )(h, p0, p1, wa, wb, b1.reshape(1, -1), w2, b2.reshape(1, -1))


def _sc_gather1(tab, idx, rows, nch):
    rpw = rows // _NW

    @functools.partial(
        pl.kernel, mesh=_mesh(),
        out_type=jax.ShapeDtypeStruct((rows, _H), jnp.float32),
        scratch_types=[pltpu.VMEM((nch, _CHUNK), jnp.int32),
                       pltpu.VMEM((_CHUNK, _H), jnp.float32),
                       pltpu.VMEM((_CHUNK, _H), jnp.float32),
                       pltpu.VMEM_SHARED((_NP, _H), jnp.float32),
                       pltpu.SemaphoreType.DMA,
                       pltpu.SemaphoreType.DMA],
    )
    def k(t_hbm, i_hbm, o_hbm, i_v, ra, rb, tab_sh, sg1, sg2):
        sid = lax.axis_index("s")
        w = lax.axis_index("c") * 16 + sid
        base = w * rpw
        st = pltpu.async_copy(t_hbm.at[pl.ds(sid * _ZR, _ZR)],
                              tab_sh.at[pl.ds(sid * _ZR, _ZR)], sg2)
        pltpu.async_copy(i_hbm.at[w], i_v, sg1).wait()
        st.wait()
        plsc.subcore_barrier()
        pltpu.async_copy(tab_sh.at[i_v.at[0]], ra, sg1)

        def body(ci, rc, rn):
            @pl.when(ci + 1 < nch)
            def _():
                pltpu.async_copy(tab_sh.at[i_v.at[ci + 1]], rn, sg1)

            pltpu.make_async_copy(tab_sh.at[i_v.at[ci]], rc, sg1).wait()
            pltpu.sync_copy(rc, o_hbm.at[pl.ds(base + ci * _CHUNK, _CHUNK)])

        @pl.loop(0, nch, step=2)
        def _(ci):
            body(ci, ra, rb)
            body(ci + 1, rb, ra)

    return k(tab, idx)


def _sc_scatter_add(m, idx, init_tab, rows, nch):
    rpw = rows // _NW

    @functools.partial(
        pl.kernel, mesh=_mesh(),
        out_type=jax.ShapeDtypeStruct((2, _NP, _H), jnp.float32),
        scratch_types=[pltpu.VMEM((nch, _CHUNK), jnp.int32),
                       pltpu.VMEM((_CHUNK, _H), jnp.float32),
                       pltpu.VMEM((_CHUNK, _H), jnp.float32),
                       pltpu.VMEM_SHARED((_NP, _H), jnp.float32),
                       pltpu.SemaphoreType.DMA],
    )
    def k(m_hbm, i_hbm, z_hbm, o_hbm, i_v, ra, rb, acc_sh, sem):
        cid = lax.axis_index("c")
        sid = lax.axis_index("s")
        w = cid * 16 + sid
        base = w * rpw
        pltpu.sync_copy(z_hbm.at[cid, pl.ds(sid * _ZR, _ZR)],
                        acc_sh.at[pl.ds(sid * _ZR, _ZR)])
        pltpu.async_copy(i_hbm.at[w], i_v, sem).wait()
        plsc.subcore_barrier()
        pltpu.async_copy(m_hbm.at[pl.ds(base, _CHUNK)], ra, sem)

        def body(ci, cur, nxt):
            @pl.when(ci + 1 < nch)
            def _():
                pltpu.async_copy(
                    m_hbm.at[pl.ds(base + (ci + 1) * _CHUNK, _CHUNK)], nxt, sem)

            pltpu.make_async_copy(
                m_hbm.at[pl.ds(base + ci * _CHUNK, _CHUNK)], cur, sem).wait()
            pltpu.sync_copy(cur, acc_sh.at[i_v.at[ci]], add=True)

        @pl.loop(0, nch, step=2)
        def _(ci):
            body(ci, ra, rb)
            body(ci + 1, rb, ra)

        plsc.subcore_barrier()
        pltpu.sync_copy(acc_sh.at[pl.ds(sid * _ZR, _ZR)],
                        o_hbm.at[cid, pl.ds(sid * _ZR, _ZR)])

    return k(m, idx, init_tab)


def kernel(x, edge_index, edge_attr, node_positions,
           enW1, enb1, enW2, enb2,
           eeW1, eeb1, eeW2, eeb2,
           mp0_eW1, mp0_eb1, mp0_eW2, mp0_eb2,
           mp0_nW1, mp0_nb1, mp0_nW2, mp0_nb2,
           mp1_eW1, mp1_eb1, mp1_eW2, mp1_eb2,
           mp1_nW1, mp1_nb1, mp1_nW2, mp1_nb2,
           deW1, deb1, deW2, deb2):
    del node_positions
    nsl = 1
    eps = _EP // nsl
    nchs = _NCH // nsl
    xp = jnp.pad(x[0], ((0, _NP - _N), (0, 0)))
    ea = jnp.pad(edge_attr[0], ((0, _EP - _E), (0, 0)))
    src = jnp.pad(edge_index[0], (0, _EP - _E))
    dstg = jnp.pad(edge_index[1], (0, _EP - _E))
    dsts = jnp.pad(edge_index[1], (0, _EP - _E), constant_values=_N)

    def _slc(a):
        return [a[s0 * eps:(s0 + 1) * eps].reshape(_NW, nchs, _CHUNK)
                for s0 in range(nsl)]

    src_s = _slc(src)
    dstg_s = _slc(dstg)
    dsts_s = _slc(dsts)
    zeros_tab = jnp.zeros((2, _NP, _H), dtype=jnp.float32)

    h = _mlp2_rows(xp, enW1, enb1, enW2, enb2)
    e_s = [_mlp2_rows(ea[s0 * eps:(s0 + 1) * eps], eeW1, eeb1, eeW2, eeb2)
           for s0 in range(nsl)]

    mps = [(mp0_eW1, mp0_eb1, mp0_eW2, mp0_eb2,
            mp0_nW1, mp0_nb1, mp0_nW2, mp0_nb2),
           (mp1_eW1, mp1_eb1, mp1_eW2, mp1_eb2,
            mp1_nW1, mp1_nb1, mp1_nW2, mp1_nb2)]
    for (ew1, eb1, ew2, eb2, nw1, nb1, nw2, nb2) in mps:
        wa, wb, wc = ew1[:_H], ew1[_H:2 * _H], ew1[2 * _H:]
        hs, hd = _dual_mm(h, wa, wb)
        g_s = [(_sc_gather1(hs, src_s[s0], eps, nchs),
                _sc_gather1(hd, dstg_s[s0], eps, nchs))
               for s0 in range(nsl)]
        m_s = [None] * nsl
        for s0 in range(nsl):
            m_s[s0], e_s[s0] = _edge_post(g_s[s0][0], g_s[s0][1], e_s[s0],
                                          wc, eb1, ew2, eb2)
        p = zeros_tab
        for s0 in range(nsl):
            p = _sc_scatter_add(m_s[s0], dsts_s[s0], p, eps, nchs)
        h = _node_post(h, p[0], p[1], nw1[:_H], nw1[_H:], nb1, nw2, nb2)

    out = _mlp2_rows(h, deW1, deb1, deW2, deb2)
    return out[:_N][None]


# speedup vs baseline: 1.0492x; 1.0492x over previous
import functools

import jax
import jax.numpy as jnp
from jax import lax
from jax.experimental import pallas as pl
from jax.experimental.pallas import tpu as pltpu
from jax.experimental.pallas import tpu_sc as plsc

_N = 10000
_E = 160000
_H = 128
_NP = 10240
_EP = 163840
_NW = 32
_CHUNK = 128
_RPW = _EP // _NW
_NCH = _RPW // _CHUNK
_ZR = _NP // 16
_BLK = 512

def _mesh():
    return plsc.VectorSubcoreMesh(core_axis_name="c", subcore_axis_name="s")


def _dot(a, b):
    return lax.dot_general(a, b, (((1,), (0,)), ((), ())),
                           preferred_element_type=jnp.float32)


def _mlp2_body(x_ref, w1_ref, b1_ref, w2_ref, b2_ref, o_ref):
    t = jnp.maximum(_dot(x_ref[...], w1_ref[...]) + b1_ref[...], 0.0)
    o_ref[...] = _dot(t, w2_ref[...]) + b2_ref[...]


def _mlp2_rows(xx, w1, b1, w2, b2):
    r, k = xx.shape
    hh = w1.shape[1]
    oo = w2.shape[1]
    return pl.pallas_call(
        _mlp2_body,
        grid=(r // _BLK,),
        in_specs=[pl.BlockSpec((_BLK, k), lambda i: (i, 0)),
                  pl.BlockSpec((k, hh), lambda i: (0, 0)),
                  pl.BlockSpec((1, hh), lambda i: (0, 0)),
                  pl.BlockSpec((hh, oo), lambda i: (0, 0)),
                  pl.BlockSpec((1, oo), lambda i: (0, 0))],
        out_specs=pl.BlockSpec((_BLK, oo), lambda i: (i, 0)),
        out_shape=jax.ShapeDtypeStruct((r, oo), jnp.float32),
        compiler_params=pltpu.CompilerParams(
            dimension_semantics=("parallel",)),
    )(xx, w1, b1.reshape(1, -1), w2, b2.reshape(1, -1))


def _enc_nodes_tab_body(x_ref, w1_ref, b1_ref, w2_ref, b2_ref,
                        wa_ref, wb_ref, h_ref, o1_ref, o2_ref):
    t = jnp.maximum(_dot(x_ref[...], w1_ref[...]) + b1_ref[...], 0.0)
    h = _dot(t, w2_ref[...]) + b2_ref[...]
    h_ref[...] = h
    o1_ref[...] = _dot(h, wa_ref[...])
    o2_ref[...] = _dot(h, wb_ref[...])


def _enc_nodes_tab(xx, w1, b1, w2, b2, wa, wb):
    r = xx.shape[0]
    return pl.pallas_call(
        _enc_nodes_tab_body,
        grid=(r // _BLK,),
        in_specs=[pl.BlockSpec((_BLK, _H), lambda i: (i, 0)),
                  pl.BlockSpec((_H, _H), lambda i: (0, 0)),
                  pl.BlockSpec((1, _H), lambda i: (0, 0)),
                  pl.BlockSpec((_H, _H), lambda i: (0, 0)),
                  pl.BlockSpec((1, _H), lambda i: (0, 0)),
                  pl.BlockSpec((_H, _H), lambda i: (0, 0)),
                  pl.BlockSpec((_H, _H), lambda i: (0, 0))],
        out_specs=[pl.BlockSpec((_BLK, _H), lambda i: (i, 0)),
                   pl.BlockSpec((_BLK, _H), lambda i: (i, 0)),
                   pl.BlockSpec((_BLK, _H), lambda i: (i, 0))],
        out_shape=[jax.ShapeDtypeStruct((r, _H), jnp.float32),
                   jax.ShapeDtypeStruct((r, _H), jnp.float32),
                   jax.ShapeDtypeStruct((r, _H), jnp.float32)],
        compiler_params=pltpu.CompilerParams(
            dimension_semantics=("parallel",)),
    )(xx, w1, b1.reshape(1, -1), w2, b2.reshape(1, -1), wa, wb)


def _edge_post_body(g1_ref, g2_ref, e_ref, wc_ref, b1_ref, w2_ref, b2_ref,
                    m_ref, eo_ref):
    e = e_ref[...]
    pre = (g1_ref[...] + g2_ref[...]
           + _dot(e, wc_ref[...]) + b1_ref[...])
    m = _dot(jnp.maximum(pre, 0.0), w2_ref[...]) + b2_ref[...]
    m_ref[...] = m
    eo_ref[...] = e + m


def _edge_post(g1, g2, e, e_off, wc, b1, w2, b2):
    r = g1.shape[0]
    return pl.pallas_call(
        _edge_post_body,
        grid=(r // _BLK,),
        in_specs=[pl.BlockSpec((_BLK, _H), lambda i: (i, 0)),
                  pl.BlockSpec((_BLK, _H), lambda i: (i, 0)),
                  pl.BlockSpec((_BLK, _H), lambda i, e_off=e_off: (i + e_off, 0)),
                  pl.BlockSpec((_H, _H), lambda i: (0, 0)),
                  pl.BlockSpec((1, _H), lambda i: (0, 0)),
                  pl.BlockSpec((_H, _H), lambda i: (0, 0)),
                  pl.BlockSpec((1, _H), lambda i: (0, 0))],
        out_specs=[pl.BlockSpec((_BLK, _H), lambda i: (i, 0)),
                   pl.BlockSpec((_BLK, _H), lambda i: (i, 0))],
        out_shape=[jax.ShapeDtypeStruct((r, _H), jnp.float32),
                   jax.ShapeDtypeStruct((r, _H), jnp.float32)],
        compiler_params=pltpu.CompilerParams(
            dimension_semantics=("parallel",)),
    )(g1, g2, e, wc, b1.reshape(1, -1), w2, b2.reshape(1, -1))


def _node_post_body(h_ref, p0_ref, p1_ref, wa_ref, wb_ref, b1_ref,
                    w2_ref, b2_ref, o_ref):
    h = h_ref[...]
    agg = p0_ref[...] + p1_ref[...]
    t = jnp.maximum(_dot(h, wa_ref[...]) + _dot(agg, wb_ref[...]) + b1_ref[...],
                    0.0)
    o_ref[...] = h + _dot(t, w2_ref[...]) + b2_ref[...]


def _node_post_tab_body(h_ref, p0_ref, p1_ref, wa_ref, wb_ref, b1_ref,
                        w2_ref, b2_ref, na_ref, nb_ref,
                        o_ref, o1_ref, o2_ref):
    h = h_ref[...]
    agg = p0_ref[...] + p1_ref[...]
    t = jnp.maximum(_dot(h, wa_ref[...]) + _dot(agg, wb_ref[...]) + b1_ref[...],
                    0.0)
    hn = h + _dot(t, w2_ref[...]) + b2_ref[...]
    o_ref[...] = hn
    o1_ref[...] = _dot(hn, na_ref[...])
    o2_ref[...] = _dot(hn, nb_ref[...])


def _node_post_tab(h, p0, p1, wa, wb, b1, w2, b2, na, nb):
    r = h.shape[0]
    return pl.pallas_call(
        _node_post_tab_body,
        grid=(r // _BLK,),
        in_specs=[pl.BlockSpec((_BLK, _H), lambda i: (i, 0)),
                  pl.BlockSpec((_BLK, _H), lambda i: (i, 0)),
                  pl.BlockSpec((_BLK, _H), lambda i: (i, 0)),
                  pl.BlockSpec((_H, _H), lambda i: (0, 0)),
                  pl.BlockSpec((_H, _H), lambda i: (0, 0)),
                  pl.BlockSpec((1, _H), lambda i: (0, 0)),
                  pl.BlockSpec((_H, _H), lambda i: (0, 0)),
                  pl.BlockSpec((1, _H), lambda i: (0, 0)),
                  pl.BlockSpec((_H, _H), lambda i: (0, 0)),
                  pl.BlockSpec((_H, _H), lambda i: (0, 0))],
        out_specs=[pl.BlockSpec((_BLK, _H), lambda i: (i, 0)),
                   pl.BlockSpec((_BLK, _H), lambda i: (i, 0)),
                   pl.BlockSpec((_BLK, _H), lambda i: (i, 0))],
        out_shape=[jax.ShapeDtypeStruct((r, _H), jnp.float32),
                   jax.ShapeDtypeStruct((r, _H), jnp.float32),
                   jax.ShapeDtypeStruct((r, _H), jnp.float32)],
        compiler_params=pltpu.CompilerParams(
            dimension_semantics=("parallel",)),
    )(h, p0, p1, wa, wb, b1.reshape(1, -1), w2, b2.reshape(1, -1), na, nb)


def _node_post(h, p0, p1, wa, wb, b1, w2, b2):
    r = h.shape[0]
    return pl.pallas_call(
        _node_post_body,
        grid=(r // _BLK,),
        in_specs=[pl.BlockSpec((_BLK, _H), lambda i: (i, 0)),
                  pl.BlockSpec((_BLK, _H), lambda i: (i, 0)),
                  pl.BlockSpec((_BLK, _H), lambda i: (i, 0)),
                  pl.BlockSpec((_H, _H), lambda i: (0, 0)),
                  pl.BlockSpec((_H, _H), lambda i: (0, 0)),
                  pl.BlockSpec((1, _H), lambda i: (0, 0)),
                  pl.BlockSpec((_H, _H), lambda i: (0, 0)),
                  pl.BlockSpec((1, _H), lambda i: (0, 0))],
        out_specs=pl.BlockSpec((_BLK, _H), lambda i: (i, 0)),
        out_shape=jax.ShapeDtypeStruct((r, _H), jnp.float32),
        compiler_params=pltpu.CompilerParams(
            dimension_semantics=("parallel",)),
    )(h, p0, p1, wa, wb, b1.reshape(1, -1), w2, b2.reshape(1, -1))


def _sc_gather1(tab, idx, rows, nch):
    rpw = rows // _NW

    @functools.partial(
        pl.kernel, mesh=_mesh(),
        out_type=jax.ShapeDtypeStruct((rows, _H), jnp.float32),
        scratch_types=[pltpu.VMEM((nch, _CHUNK), jnp.int32),
                       pltpu.VMEM((_CHUNK, _H), jnp.float32),
                       pltpu.VMEM((_CHUNK, _H), jnp.float32),
                       pltpu.VMEM_SHARED((_NP, _H), jnp.float32),
                       pltpu.SemaphoreType.DMA,
                       pltpu.SemaphoreType.DMA],
    )
    def k(t_hbm, i_hbm, o_hbm, i_v, ra, rb, tab_sh, sg1, sg2):
        sid = lax.axis_index("s")
        w = lax.axis_index("c") * 16 + sid
        base = w * rpw
        st = pltpu.async_copy(t_hbm.at[pl.ds(sid * _ZR, _ZR)],
                              tab_sh.at[pl.ds(sid * _ZR, _ZR)], sg2)
        pltpu.async_copy(i_hbm.at[w], i_v, sg1).wait()
        st.wait()
        plsc.subcore_barrier()
        pltpu.async_copy(tab_sh.at[i_v.at[0]], ra, sg1)

        def body(ci, rc, rn):
            @pl.when(ci + 1 < nch)
            def _():
                pltpu.async_copy(tab_sh.at[i_v.at[ci + 1]], rn, sg1)

            pltpu.make_async_copy(tab_sh.at[i_v.at[ci]], rc, sg1).wait()
            pltpu.sync_copy(rc, o_hbm.at[pl.ds(base + ci * _CHUNK, _CHUNK)])

        @pl.loop(0, nch, step=2)
        def _(ci):
            body(ci, ra, rb)
            body(ci + 1, rb, ra)

    return k(tab, idx)


def _sc_scatter_add(m, idx, init_tab, rows, nch):
    rpw = rows // _NW

    @functools.partial(
        pl.kernel, mesh=_mesh(),
        out_type=jax.ShapeDtypeStruct((2, _NP, _H), jnp.float32),
        scratch_types=[pltpu.VMEM((nch, _CHUNK), jnp.int32),
                       pltpu.VMEM((_CHUNK, _H), jnp.float32),
                       pltpu.VMEM((_CHUNK, _H), jnp.float32),
                       pltpu.VMEM_SHARED((_NP, _H), jnp.float32),
                       pltpu.SemaphoreType.DMA],
    )
    def k(m_hbm, i_hbm, z_hbm, o_hbm, i_v, ra, rb, acc_sh, sem):
        cid = lax.axis_index("c")
        sid = lax.axis_index("s")
        w = cid * 16 + sid
        base = w * rpw
        pltpu.sync_copy(z_hbm.at[cid, pl.ds(sid * _ZR, _ZR)],
                        acc_sh.at[pl.ds(sid * _ZR, _ZR)])
        pltpu.async_copy(i_hbm.at[w], i_v, sem).wait()
        plsc.subcore_barrier()
        pltpu.async_copy(m_hbm.at[pl.ds(base, _CHUNK)], ra, sem)

        def body(ci, cur, nxt):
            @pl.when(ci + 1 < nch)
            def _():
                pltpu.async_copy(
                    m_hbm.at[pl.ds(base + (ci + 1) * _CHUNK, _CHUNK)], nxt, sem)

            pltpu.make_async_copy(
                m_hbm.at[pl.ds(base + ci * _CHUNK, _CHUNK)], cur, sem).wait()
            pltpu.sync_copy(cur, acc_sh.at[i_v.at[ci]], add=True)

        @pl.loop(0, nch, step=2)
        def _(ci):
            body(ci, ra, rb)
            body(ci + 1, rb, ra)

        plsc.subcore_barrier()
        pltpu.sync_copy(acc_sh.at[pl.ds(sid * _ZR, _ZR)],
                        o_hbm.at[cid, pl.ds(sid * _ZR, _ZR)])

    return k(m, idx, init_tab)


def kernel(x, edge_index, edge_attr, node_positions,
           enW1, enb1, enW2, enb2,
           eeW1, eeb1, eeW2, eeb2,
           mp0_eW1, mp0_eb1, mp0_eW2, mp0_eb2,
           mp0_nW1, mp0_nb1, mp0_nW2, mp0_nb2,
           mp1_eW1, mp1_eb1, mp1_eW2, mp1_eb2,
           mp1_nW1, mp1_nb1, mp1_nW2, mp1_nb2,
           deW1, deb1, deW2, deb2):
    del node_positions
    nsl = 2
    eps = _EP // nsl
    nchs = _NCH // nsl
    xp = jnp.pad(x[0], ((0, _NP - _N), (0, 0)))
    ea = jnp.pad(edge_attr[0], ((0, _EP - _E), (0, 0)))
    src = jnp.pad(edge_index[0], (0, _EP - _E))
    dstg = jnp.pad(edge_index[1], (0, _EP - _E))
    dsts = jnp.pad(edge_index[1], (0, _EP - _E), constant_values=_N)

    def _slc(a):
        return [a[s0 * eps:(s0 + 1) * eps].reshape(_NW, nchs, _CHUNK)
                for s0 in range(nsl)]

    src_s = _slc(src)
    dstg_s = _slc(dstg)
    dsts_s = _slc(dsts)
    zeros_tab = jnp.zeros((2, _NP, _H), dtype=jnp.float32)

    e_full = _mlp2_rows(ea, eeW1, eeb1, eeW2, eeb2)
    e_s = [e_full] * nsl
    e_off = [s0 * (eps // _BLK) for s0 in range(nsl)]

    h, hs, hd = _enc_nodes_tab(xp, enW1, enb1, enW2, enb2,
                               mp0_eW1[:_H], mp0_eW1[_H:2 * _H])

    mps = [(mp0_eW1, mp0_eb1, mp0_eW2, mp0_eb2,
            mp0_nW1, mp0_nb1, mp0_nW2, mp0_nb2),
           (mp1_eW1, mp1_eb1, mp1_eW2, mp1_eb2,
            mp1_nW1, mp1_nb1, mp1_nW2, mp1_nb2)]
    for li, (ew1, eb1, ew2, eb2, nw1, nb1, nw2, nb2) in enumerate(mps):
        wc = ew1[2 * _H:]
        g_s = [(_sc_gather1(hs, src_s[s0], eps, nchs),
                _sc_gather1(hd, dstg_s[s0], eps, nchs))
               for s0 in range(nsl)]
        m_s = [None] * nsl
        for s0 in range(nsl):
            m_s[s0], e_s[s0] = _edge_post(g_s[s0][0], g_s[s0][1], e_s[s0],
                                          e_off[s0], wc, eb1, ew2, eb2)
        e_off = [0] * nsl
        p = zeros_tab
        for s0 in range(nsl):
            p = _sc_scatter_add(m_s[s0], dsts_s[s0], p, eps, nchs)
        if li == 0:
            h, hs, hd = _node_post_tab(h, p[0], p[1], nw1[:_H], nw1[_H:],
                                       nb1, nw2, nb2,
                                       mp1_eW1[:_H], mp1_eW1[_H:2 * _H])
        else:
            h = _node_post(h, p[0], p[1], nw1[:_H], nw1[_H:], nb1, nw2, nb2)

    out = _mlp2_rows(h, deW1, deb1, deW2, deb2)
    return out[:_N][None]
